# baseline (speedup 1.0000x reference)
import jax
import jax.numpy as jnp
from jax.experimental import pallas as pl
from jax.experimental.pallas import tpu as pltpu

_F = 4
_TS = 131072


def _round_up(v, m):
    return ((v + m - 1) // m) * m


def _body(x_ref, wb_ref, o_ref):
    xs = [x_ref[i, 0, :] for i in range(_F)]
    for j in range(_F):
        y = wb_ref[16 + j]
        for i in range(_F):
            y = y + wb_ref[4 * i + j] * xs[i]
        o_ref[j, 0, :] = jnp.exp(-50.0 * (y * y))


def kernel(x, w, b):
    B, f_in = x.shape
    f_out = w.shape[1]
    assert f_in == _F and f_out == _F

    xt = x.T
    pBS = _round_up(B, _TS)
    if pBS != B:
        xt = jnp.pad(xt, ((0, 0), (0, pBS - B)))
    xt3 = xt.reshape(_F, 1, pBS)

    wb = jnp.concatenate([w.reshape(-1), b])

    grid = (pBS // _TS,)

    out_t = pl.pallas_call(
        _body,
        out_shape=jax.ShapeDtypeStruct((_F, 1, pBS), jnp.float32),
        grid=grid,
        in_specs=[
            pl.BlockSpec((_F, 1, _TS), lambda i: (0, 0, i)),
            pl.BlockSpec(memory_space=pltpu.SMEM),
        ],
        out_specs=pl.BlockSpec((_F, 1, _TS), lambda i: (0, 0, i)),
        compiler_params=pltpu.CompilerParams(
            dimension_semantics=("parallel",),
            vmem_limit_bytes=48 * 1024 * 1024,
        ),
        cost_estimate=pl.CostEstimate(
            flops=2 * pBS * _F * _F,
            transcendentals=pBS * _F,
            bytes_accessed=2 * pBS * _F * 4,
        ),
    )(xt3, wb)

    return out_t.reshape(_F, pBS)[:, :B].T


# speedup vs baseline: 1.4502x; 1.4502x over previous
import jax
import jax.numpy as jnp
from jax.experimental import pallas as pl
from jax.experimental.pallas import tpu as pltpu

_F = 4
_TS = 32768


def _round_up(v, m):
    return ((v + m - 1) // m) * m


def _split_hi_lo(a):
    bits = jax.lax.bitcast_convert_type(a, jnp.uint32)
    hi = jax.lax.bitcast_convert_type(
        bits & jnp.uint32(0xFFFF0000), jnp.float32)
    return hi, a - hi


def _body(x_ref, a_ref, o_ref):
    xb = x_ref[...]
    hi, lo = _split_hi_lo(xb)
    hi = hi.astype(jnp.bfloat16)
    lo = lo.astype(jnp.bfloat16)
    ones = jnp.ones_like(hi[0:2])
    zero = jnp.zeros_like(ones)
    rhs = jnp.concatenate([hi, lo, hi, ones, zero], axis=0)
    y = jnp.dot(a_ref[...], rhs, preferred_element_type=jnp.float32)
    y4 = y[0:4]
    o_ref[...] = jnp.exp(-50.0 * (y4 * y4))


def kernel(x, w, b):
    B, f_in = x.shape
    f_out = w.shape[1]
    assert f_in == _F and f_out == _F

    xt = x.T
    pBS = _round_up(B, _TS)
    if pBS != B:
        xt = jnp.pad(xt, ((0, 0), (0, pBS - B)))

    wh, wl = _split_hi_lo(w)
    bh, bl = _split_hi_lo(b)
    a16 = jnp.concatenate(
        [wh.T, wh.T, wl.T, bh.reshape(_F, 1), bl.reshape(_F, 1),
         jnp.zeros((_F, 2), jnp.float32)], axis=1)
    a16 = jnp.concatenate([a16, jnp.zeros((4, 16), jnp.float32)], axis=0)
    a16 = a16.astype(jnp.bfloat16)

    grid = (pBS // _TS,)

    out_t = pl.pallas_call(
        _body,
        out_shape=jax.ShapeDtypeStruct((_F, pBS), jnp.float32),
        grid=grid,
        in_specs=[
            pl.BlockSpec((_F, _TS), lambda i: (0, i)),
            pl.BlockSpec((8, 16), lambda i: (0, 0)),
        ],
        out_specs=pl.BlockSpec((_F, _TS), lambda i: (0, i)),
        compiler_params=pltpu.CompilerParams(
            dimension_semantics=("parallel",),
            vmem_limit_bytes=56 * 1024 * 1024,
        ),
        cost_estimate=pl.CostEstimate(
            flops=2 * pBS * 16 * 8,
            transcendentals=pBS * _F,
            bytes_accessed=2 * pBS * _F * 4,
        ),
    )(xt, a16)

    return out_t[:, :B].T


# speedup vs baseline: 2.3728x; 1.6362x over previous
import jax
import jax.numpy as jnp
from jax.experimental import pallas as pl
from jax.experimental.pallas import tpu as pltpu

_F = 4
_TS = 262144


def _round_up(v, m):
    return ((v + m - 1) // m) * m


def _split_hi_lo(a):
    bits = jax.lax.bitcast_convert_type(a, jnp.uint32)
    hi = jax.lax.bitcast_convert_type(
        bits & jnp.uint32(0xFFFF0000), jnp.float32)
    return hi, a - hi


def _body(x_ref, a_ref, o_ref):
    xb = x_ref[...]
    hi, lo = _split_hi_lo(xb)
    hi = hi.astype(jnp.bfloat16)
    lo = lo.astype(jnp.bfloat16)
    ones = jnp.ones_like(hi[0:2])
    zero = jnp.zeros_like(ones)
    rhs = jnp.concatenate([hi, lo, hi, ones, zero], axis=0)
    y = jnp.dot(a_ref[...], rhs, preferred_element_type=jnp.float32)
    y4 = y[0:4]
    o_ref[...] = jnp.exp(-50.0 * (y4 * y4))


def kernel(x, w, b):
    B, f_in = x.shape
    f_out = w.shape[1]
    assert f_in == _F and f_out == _F

    xt = x.T
    pBS = _round_up(B, _TS)
    if pBS != B:
        xt = jnp.pad(xt, ((0, 0), (0, pBS - B)))

    wh, wl = _split_hi_lo(w)
    bh, bl = _split_hi_lo(b)
    a16 = jnp.concatenate(
        [wh.T, wh.T, wl.T, bh.reshape(_F, 1), bl.reshape(_F, 1),
         jnp.zeros((_F, 2), jnp.float32)], axis=1)
    a16 = jnp.concatenate([a16, jnp.zeros((4, 16), jnp.float32)], axis=0)
    a16 = a16.astype(jnp.bfloat16)

    grid = (pBS // _TS,)

    out_t = pl.pallas_call(
        _body,
        out_shape=jax.ShapeDtypeStruct((_F, pBS), jnp.float32),
        grid=grid,
        in_specs=[
            pl.BlockSpec((_F, _TS), lambda i: (0, i)),
            pl.BlockSpec((8, 16), lambda i: (0, 0)),
        ],
        out_specs=pl.BlockSpec((_F, _TS), lambda i: (0, i)),
        compiler_params=pltpu.CompilerParams(
            dimension_semantics=("parallel",),
            vmem_limit_bytes=56 * 1024 * 1024,
        ),
        cost_estimate=pl.CostEstimate(
            flops=2 * pBS * 16 * 8,
            transcendentals=pBS * _F,
            bytes_accessed=2 * pBS * _F * 4,
        ),
    )(xt, a16)

    return out_t[:, :B].T
